# baseline (speedup 1.0000x reference)
import functools

import jax
import jax.numpy as jnp
from jax import lax
from jax.experimental import pallas as pl
from jax.experimental.pallas import tpu as pltpu
from jax.experimental.pallas import tpu_sc as plsc

_NC = 2
_NS = 16
_NW = _NC * _NS
_L = 16
_CHUNK = 100
_NBUF = 4


def _sc_pooled_sum(x2, table):
  twoB, chunk = x2.shape
  assert chunk == _CHUNK
  B = twoB // 2
  D = table.shape[1]
  assert D % _L == 0 and B % _NW == 0
  b_per_w = B // _NW
  c_per_w = 2 * b_per_w
  nd = D // _L

  mesh = plsc.VectorSubcoreMesh(
      core_axis_name="c", subcore_axis_name="s",
      num_cores=_NC, num_subcores=_NS)

  @functools.partial(
      pl.kernel,
      out_type=jax.ShapeDtypeStruct((B, D), jnp.float32),
      mesh=mesh,
      scratch_types=[
          pltpu.VMEM((c_per_w, _CHUNK), jnp.int32),
          pltpu.VMEM((_NBUF, _CHUNK, D), jnp.float32),
          pltpu.VMEM((b_per_w, D), jnp.float32),
      ] + [pltpu.SemaphoreType.DMA] * _NBUF,
      compiler_params=pltpu.CompilerParams(use_tc_tiling_on_sc=False),
  )
  def k(x2_hbm, table_hbm, out_hbm, idx_v, rows_v, out_v, *sems):
    wid = lax.axis_index("s") * _NC + lax.axis_index("c")
    cbase = wid * c_per_w
    bbase = wid * b_per_w

    pltpu.sync_copy(x2_hbm.at[pl.ds(cbase, c_per_w)], idx_v)

    for k0 in range(_NBUF):
      pltpu.async_copy(table_hbm.at[idx_v.at[k0]], rows_v.at[k0], sems[k0])

    def accum_chunk(buf, accs):
      def body(r4, accs):
        accs = list(accs)
        for u in range(4):
          r = r4 * 4 + u
          for d in range(nd):
            accs[d] = accs[d] + rows_v[buf, r, d * _L:(d + 1) * _L]
        return tuple(accs)
      return lax.fori_loop(0, _CHUNK // 4, body, accs)

    def pair_body(p, carry):
      for half in range(2):
        i = 2 * p + half
        accs = tuple(jnp.zeros((_L,), jnp.float32) for _ in range(nd))
        for k1 in (2 * half, 2 * half + 1):
          c = 4 * p + k1
          pltpu.make_async_copy(
              table_hbm.at[idx_v.at[c]], rows_v.at[k1], sems[k1]).wait()
          accs = accum_chunk(k1, accs)

          @pl.when(c + _NBUF < c_per_w)
          def _():
            pltpu.async_copy(
                table_hbm.at[idx_v.at[c + _NBUF]], rows_v.at[k1], sems[k1])

        for d in range(nd):
          out_v[i, d * _L:(d + 1) * _L] = accs[d]
      return carry

    lax.fori_loop(0, b_per_w // 2, pair_body, 0)
    pltpu.sync_copy(out_v, out_hbm.at[pl.ds(bbase, b_per_w)])

  return k(x2, table)


def _tc_head(x, pooled_sum, fc_wt, fc_b2):
  B, S = x.shape
  D = pooled_sum.shape[1]
  C = fc_wt.shape[1]
  BLK = 256
  assert B % BLK == 0

  def body(x_ref, ps_ref, w_ref, b_ref, out_ref):
    cnt = jnp.sum((x_ref[...] != 0).astype(jnp.float32), axis=1,
                  keepdims=True)
    pooled = jnp.maximum(ps_ref[...] / cnt, 0.0)
    out_ref[...] = lax.dot_general(
        pooled, w_ref[...], (((1,), (0,)), ((), ())),
        preferred_element_type=jnp.float32) + b_ref[...]

  return pl.pallas_call(
      body,
      grid=(B // BLK,),
      in_specs=[
          pl.BlockSpec((BLK, S), lambda i: (i, 0)),
          pl.BlockSpec((BLK, D), lambda i: (i, 0)),
          pl.BlockSpec((D, C), lambda i: (0, 0)),
          pl.BlockSpec((1, C), lambda i: (0, 0)),
      ],
      out_specs=pl.BlockSpec((BLK, C), lambda i: (i, 0)),
      out_shape=jax.ShapeDtypeStruct((B, C), jnp.float32),
  )(x, pooled_sum, fc_wt, fc_b2)


def kernel(x, table, fc_w, fc_b):
  B, S = x.shape
  x = x.astype(jnp.int32)
  x2 = x.reshape(2 * B, S // 2)
  pooled_sum = _sc_pooled_sum(x2, table)
  return _tc_head(x, pooled_sum, fc_w.T, fc_b.reshape(1, -1))


# speedup vs baseline: 1.5678x; 1.5678x over previous
import functools

import jax
import jax.numpy as jnp
from jax import lax
from jax.experimental import pallas as pl
from jax.experimental.pallas import tpu as pltpu
from jax.experimental.pallas import tpu_sc as plsc

_NC = 2
_NS = 16
_NW = _NC * _NS
_L = 16
_CHUNK = 100
_NBUF = 4


def _sc_pooled_sum(x2, table):
  twoB, chunk = x2.shape
  assert chunk == _CHUNK
  B = twoB // 2
  D = table.shape[1]
  assert D % _L == 0 and B % _NW == 0
  b_per_w = B // _NW
  c_per_w = 2 * b_per_w
  nd = D // _L

  mesh = plsc.VectorSubcoreMesh(
      core_axis_name="c", subcore_axis_name="s",
      num_cores=_NC, num_subcores=_NS)

  @functools.partial(
      pl.kernel,
      out_type=jax.ShapeDtypeStruct((B, D), jnp.float32),
      mesh=mesh,
      scratch_types=[
          pltpu.VMEM((c_per_w, _CHUNK), jnp.int32),
          pltpu.VMEM((_NBUF, _CHUNK, D), jnp.float32),
          pltpu.VMEM((b_per_w, D), jnp.float32),
      ] + [pltpu.SemaphoreType.DMA] * _NBUF,
      compiler_params=pltpu.CompilerParams(use_tc_tiling_on_sc=False),
  )
  def k(x2_hbm, table_hbm, out_hbm, idx_v, rows_v, out_v, *sems):
    wid = lax.axis_index("s") * _NC + lax.axis_index("c")
    cbase = wid * c_per_w
    bbase = wid * b_per_w

    pltpu.sync_copy(x2_hbm.at[pl.ds(cbase, c_per_w)], idx_v)

    for k0 in range(_NBUF):
      pltpu.async_copy(table_hbm.at[idx_v.at[k0]], rows_v.at[k0], sems[k0])

    def accum_chunk(buf, accs):
      def body(r4, accs):
        accs = list(accs)
        for u in range(4):
          r = r4 * 4 + u
          for d in range(nd):
            accs[d] = accs[d] + rows_v[buf, r, d * _L:(d + 1) * _L]
        return tuple(accs)
      return lax.fori_loop(0, _CHUNK // 4, body, accs)

    def pair_body(p, carry):
      for half in range(2):
        i = 2 * p + half
        accs = tuple(jnp.zeros((_L,), jnp.float32) for _ in range(nd))
        for k1 in (2 * half, 2 * half + 1):
          c = 4 * p + k1
          pltpu.make_async_copy(
              table_hbm.at[idx_v.at[c]], rows_v.at[k1], sems[k1]).wait()
          accs = accum_chunk(k1, accs)

          @pl.when(c + _NBUF < c_per_w)
          def _():
            pltpu.async_copy(
                table_hbm.at[idx_v.at[c + _NBUF]], rows_v.at[k1], sems[k1])

        for d in range(nd):
          out_v[i, d * _L:(d + 1) * _L] = accs[d]
      return carry

    lax.fori_loop(0, b_per_w // 2, pair_body, 0)
    pltpu.sync_copy(out_v, out_hbm.at[pl.ds(bbase, b_per_w)])

  return k(x2, table)


def _tc_relayout(tT):
  D, V = tT.shape
  CB = 2048
  NBC = pl.cdiv(V, CB)
  NB = pl.cdiv(NBC, 2)

  def body(lo_ref, hi_ref, out_ref):
    out_ref[:, :D] = lo_ref[...].T
    out_ref[:, D:] = hi_ref[...].T

  return pl.pallas_call(
      body,
      grid=(NB,),
      in_specs=[
          pl.BlockSpec((D, CB), lambda i: (0, 2 * i)),
          pl.BlockSpec((D, CB), lambda i: (0, jnp.minimum(2 * i + 1, NBC - 1))),
      ],
      out_specs=pl.BlockSpec((CB, 2 * D), lambda i: (i, 0)),
      out_shape=jax.ShapeDtypeStruct((NB * CB, 2 * D), jnp.float32),
  )(tT, tT)


def _tc_head(x, pooled_sum, fc_wt, fc_b2):
  B, S = x.shape
  D = pooled_sum.shape[1]
  C = fc_wt.shape[1]
  BLK = 256
  assert B % BLK == 0

  def body(x_ref, ps_ref, w_ref, b_ref, out_ref):
    cnt = jnp.sum((x_ref[...] != 0).astype(jnp.float32), axis=1,
                  keepdims=True)
    pooled = jnp.maximum(ps_ref[...] / cnt, 0.0)
    out_ref[...] = lax.dot_general(
        pooled, w_ref[...], (((1,), (0,)), ((), ())),
        preferred_element_type=jnp.float32) + b_ref[...]

  return pl.pallas_call(
      body,
      grid=(B // BLK,),
      in_specs=[
          pl.BlockSpec((BLK, S), lambda i: (i, 0)),
          pl.BlockSpec((BLK, D), lambda i: (i, 0)),
          pl.BlockSpec((D, C), lambda i: (0, 0)),
          pl.BlockSpec((1, C), lambda i: (0, 0)),
      ],
      out_specs=pl.BlockSpec((BLK, C), lambda i: (i, 0)),
      out_shape=jax.ShapeDtypeStruct((B, C), jnp.float32),
  )(x, pooled_sum, fc_wt, fc_b2)


def kernel(x, table, fc_w, fc_b):
  B, S = x.shape
  V, D = table.shape
  x = x.astype(jnp.int32)
  q, r = x >> 11, x & 2047
  px = (((q >> 1) << 11) + r) * 2 + (q & 1)
  x2 = px.reshape(2 * B, S // 2)
  tableP = _tc_relayout(table.T)
  tableL = tableP.reshape(tableP.shape[0] * 2, D)
  pooled_sum = _sc_pooled_sum(x2, tableL)
  return _tc_head(x, pooled_sum, fc_w.T, fc_b.reshape(1, -1))


# speedup vs baseline: 1.7680x; 1.1277x over previous
import functools

import jax
import jax.numpy as jnp
from jax import lax
from jax.experimental import pallas as pl
from jax.experimental.pallas import tpu as pltpu
from jax.experimental.pallas import tpu_sc as plsc

_NC = 2
_NS = 16
_NW = _NC * _NS
_L = 16
_CHUNK = 100
_NBUF = 4


def _sc_pooled_sum(x2, table):
  twoB, chunk = x2.shape
  assert chunk == _CHUNK
  B = twoB // 2
  D = table.shape[1]
  assert D % _L == 0 and B % _NW == 0
  b_per_w = B // _NW
  c_per_w = 2 * b_per_w
  nd = D // _L

  mesh = plsc.VectorSubcoreMesh(
      core_axis_name="c", subcore_axis_name="s",
      num_cores=_NC, num_subcores=_NS)

  @functools.partial(
      pl.kernel,
      out_type=jax.ShapeDtypeStruct((B, D), jnp.float32),
      mesh=mesh,
      scratch_types=[
          pltpu.VMEM((c_per_w, _CHUNK), jnp.int32),
          pltpu.VMEM((_NBUF, _CHUNK, D), jnp.float32),
          pltpu.VMEM((b_per_w, D), jnp.float32),
      ] + [pltpu.SemaphoreType.DMA] * _NBUF,
      compiler_params=pltpu.CompilerParams(use_tc_tiling_on_sc=False),
  )
  def k(x2_hbm, table_hbm, out_hbm, idx_v, rows_v, out_v, *sems):
    wid = lax.axis_index("s") * _NC + lax.axis_index("c")
    cbase = wid * c_per_w
    bbase = wid * b_per_w

    pltpu.sync_copy(x2_hbm.at[pl.ds(cbase, c_per_w)], idx_v)

    for k0 in range(_NBUF):
      pltpu.async_copy(table_hbm.at[idx_v.at[k0]], rows_v.at[k0], sems[k0])

    def accum_chunk(buf, accs):
      def body(r4, accs):
        accs = list(accs)
        for u in range(4):
          r = r4 * 4 + u
          for d in range(nd):
            accs[d] = accs[d] + rows_v[buf, r, d * _L:(d + 1) * _L]
        return tuple(accs)
      return lax.fori_loop(0, _CHUNK // 4, body, accs)

    def pair_body(p, carry):
      for half in range(2):
        i = 2 * p + half
        accs = tuple(jnp.zeros((_L,), jnp.float32) for _ in range(nd))
        for k1 in (2 * half, 2 * half + 1):
          c = 4 * p + k1
          pltpu.make_async_copy(
              table_hbm.at[idx_v.at[c]], rows_v.at[k1], sems[k1]).wait()
          accs = accum_chunk(k1, accs)

          @pl.when(c + _NBUF < c_per_w)
          def _():
            pltpu.async_copy(
                table_hbm.at[idx_v.at[c + _NBUF]], rows_v.at[k1], sems[k1])

        for d in range(nd):
          out_v[i, d * _L:(d + 1) * _L] = accs[d]
      return carry

    lax.fori_loop(0, b_per_w // 2, pair_body, 0)
    pltpu.sync_copy(out_v, out_hbm.at[pl.ds(bbase, b_per_w)])

  return k(x2, table)


def _tc_relayout(tT):
  D, V = tT.shape
  CB = 2048
  NB = pl.cdiv(V, 2 * CB)

  def body(in_ref, out_ref):
    t = in_ref[...]
    stacked = jnp.concatenate([t[:, :CB], t[:, CB:]], axis=0)
    out_ref[...] = stacked.T

  return pl.pallas_call(
      body,
      grid=(NB,),
      in_specs=[pl.BlockSpec((D, 2 * CB), lambda i: (0, i))],
      out_specs=pl.BlockSpec((CB, 2 * D), lambda i: (i, 0)),
      out_shape=jax.ShapeDtypeStruct((NB * CB, 2 * D), jnp.float32),
  )(tT)


def _tc_head(x, pooled_sum, fc_wt, fc_b2):
  B, S = x.shape
  D = pooled_sum.shape[1]
  C = fc_wt.shape[1]
  BLK = 256
  assert B % BLK == 0

  def body(x_ref, ps_ref, w_ref, b_ref, out_ref):
    cnt = jnp.sum((x_ref[...] != 0).astype(jnp.float32), axis=1,
                  keepdims=True)
    pooled = jnp.maximum(ps_ref[...] / cnt, 0.0)
    out_ref[...] = lax.dot_general(
        pooled, w_ref[...], (((1,), (0,)), ((), ())),
        preferred_element_type=jnp.float32) + b_ref[...]

  return pl.pallas_call(
      body,
      grid=(B // BLK,),
      in_specs=[
          pl.BlockSpec((BLK, S), lambda i: (i, 0)),
          pl.BlockSpec((BLK, D), lambda i: (i, 0)),
          pl.BlockSpec((D, C), lambda i: (0, 0)),
          pl.BlockSpec((1, C), lambda i: (0, 0)),
      ],
      out_specs=pl.BlockSpec((BLK, C), lambda i: (i, 0)),
      out_shape=jax.ShapeDtypeStruct((B, C), jnp.float32),
  )(x, pooled_sum, fc_wt, fc_b2)


def kernel(x, table, fc_w, fc_b):
  B, S = x.shape
  V, D = table.shape
  x = x.astype(jnp.int32)
  q, r = x >> 11, x & 2047
  px = (((q >> 1) << 11) + r) * 2 + (q & 1)
  x2 = px.reshape(2 * B, S // 2)
  tableP = _tc_relayout(table.T)
  tableL = tableP.reshape(tableP.shape[0] * 2, D)
  pooled_sum = _sc_pooled_sum(x2, tableL)
  return _tc_head(x, pooled_sum, fc_w.T, fc_b.reshape(1, -1))


# speedup vs baseline: 2.1748x; 1.2301x over previous
import functools

import jax
import jax.numpy as jnp
from jax import lax
from jax.experimental import pallas as pl
from jax.experimental.pallas import tpu as pltpu
from jax.experimental.pallas import tpu_sc as plsc

_NC = 2
_NS = 16
_NW = _NC * _NS
_L = 16
_CHUNK = 100
_NBUF = 4
_CBREL = 4096


def _sc_pooled_sum(x2, table):
  twoB, chunk = x2.shape
  assert chunk == _CHUNK
  B = twoB // 2
  D = table.shape[1]
  assert D % _L == 0 and B % _NW == 0
  b_per_w = B // _NW
  c_per_w = 2 * b_per_w
  nd = D // _L

  mesh = plsc.VectorSubcoreMesh(
      core_axis_name="c", subcore_axis_name="s",
      num_cores=_NC, num_subcores=_NS)

  @functools.partial(
      pl.kernel,
      out_type=jax.ShapeDtypeStruct((B, D), jnp.float32),
      mesh=mesh,
      scratch_types=[
          pltpu.VMEM((c_per_w, _CHUNK), jnp.int32),
          pltpu.VMEM((_NBUF, _CHUNK, D), jnp.float32),
          pltpu.VMEM((b_per_w, D), jnp.float32),
      ] + [pltpu.SemaphoreType.DMA] * _NBUF,
      compiler_params=pltpu.CompilerParams(use_tc_tiling_on_sc=False),
  )
  def k(x2_hbm, table_hbm, out_hbm, idx_v, rows_v, out_v, *sems):
    wid = lax.axis_index("s") * _NC + lax.axis_index("c")
    cbase = wid * c_per_w
    bbase = wid * b_per_w

    pltpu.sync_copy(x2_hbm.at[pl.ds(cbase, c_per_w)], idx_v)

    for k0 in range(_NBUF):
      pltpu.async_copy(table_hbm.at[idx_v.at[k0]], rows_v.at[k0], sems[k0])

    def accum_chunk(buf, accs):
      def body(r4, accs):
        accs = list(accs)
        for u in range(4):
          r = r4 * 4 + u
          for d in range(nd):
            accs[d] = accs[d] + rows_v[buf, r, d * _L:(d + 1) * _L]
        return tuple(accs)
      return lax.fori_loop(0, _CHUNK // 4, body, accs)

    def pair_body(p, carry):
      for half in range(2):
        i = 2 * p + half
        accs = tuple(jnp.zeros((_L,), jnp.float32) for _ in range(nd))
        for k1 in (2 * half, 2 * half + 1):
          c = 4 * p + k1
          pltpu.make_async_copy(
              table_hbm.at[idx_v.at[c]], rows_v.at[k1], sems[k1]).wait()
          accs = accum_chunk(k1, accs)

          @pl.when(c + _NBUF < c_per_w)
          def _():
            pltpu.async_copy(
                table_hbm.at[idx_v.at[c + _NBUF]], rows_v.at[k1], sems[k1])

        for d in range(nd):
          out_v[i, d * _L:(d + 1) * _L] = accs[d]
      return carry

    lax.fori_loop(0, b_per_w // 2, pair_body, 0)
    pltpu.sync_copy(out_v, out_hbm.at[pl.ds(bbase, b_per_w)])

  return k(x2, table)


def _tc_relayout(tT):
  D, V = tT.shape
  CB = _CBREL
  NB = pl.cdiv(V, 2 * CB)

  def body(in_ref, out_ref):
    t = in_ref[...]
    stacked = jnp.concatenate([t[:, :CB], t[:, CB:]], axis=0)
    out_ref[...] = stacked.T

  return pl.pallas_call(
      body,
      grid=(NB,),
      in_specs=[pl.BlockSpec((D, 2 * CB), lambda i: (0, i))],
      out_specs=pl.BlockSpec((CB, 2 * D), lambda i: (i, 0)),
      out_shape=jax.ShapeDtypeStruct((NB * CB, 2 * D), jnp.float32),
  )(tT)


def _tc_head(x, pooled_sum, fc_wt, fc_b2):
  B, S = x.shape
  D = pooled_sum.shape[1]
  C = fc_wt.shape[1]
  BLK = 256
  assert B % BLK == 0

  def body(x_ref, ps_ref, w_ref, b_ref, out_ref):
    cnt = jnp.sum((x_ref[...] != 0).astype(jnp.float32), axis=1,
                  keepdims=True)
    pooled = jnp.maximum(ps_ref[...] / cnt, 0.0)
    out_ref[...] = lax.dot_general(
        pooled, w_ref[...], (((1,), (0,)), ((), ())),
        preferred_element_type=jnp.float32) + b_ref[...]

  return pl.pallas_call(
      body,
      grid=(B // BLK,),
      in_specs=[
          pl.BlockSpec((BLK, S), lambda i: (i, 0)),
          pl.BlockSpec((BLK, D), lambda i: (i, 0)),
          pl.BlockSpec((D, C), lambda i: (0, 0)),
          pl.BlockSpec((1, C), lambda i: (0, 0)),
      ],
      out_specs=pl.BlockSpec((BLK, C), lambda i: (i, 0)),
      out_shape=jax.ShapeDtypeStruct((B, C), jnp.float32),
  )(x, pooled_sum, fc_wt, fc_b2)


def kernel(x, table, fc_w, fc_b):
  B, S = x.shape
  V, D = table.shape
  x = x.astype(jnp.int32)
  shift = _CBREL.bit_length() - 1
  q, r = x >> shift, x & (_CBREL - 1)
  px = (((q >> 1) << shift) + r) * 2 + (q & 1)
  x2 = px.reshape(2 * B, S // 2)
  tableT, x2 = jax.lax.optimization_barrier((table.T, x2))
  tableP = _tc_relayout(tableT)
  tableL = tableP.reshape(tableP.shape[0] * 2, D)
  pooled_sum = _sc_pooled_sum(x2, tableL)
  return _tc_head(x, pooled_sum, fc_w.T, fc_b.reshape(1, -1))


# speedup vs baseline: 2.3843x; 1.0963x over previous
import functools

import jax
import jax.numpy as jnp
from jax import lax
from jax.experimental import pallas as pl
from jax.experimental.pallas import tpu as pltpu
from jax.experimental.pallas import tpu_sc as plsc

_NC = 2
_NS = 16
_NW = _NC * _NS
_L = 16
_CHUNK = 100
_NBUF = 4
_CBREL = 8192


def _sc_pooled_sum(x2, table):
  twoB, chunk = x2.shape
  assert chunk == _CHUNK
  B = twoB // 2
  D = table.shape[1]
  assert D % _L == 0 and B % _NW == 0
  b_per_w = B // _NW
  c_per_w = 2 * b_per_w
  nd = D // _L

  mesh = plsc.VectorSubcoreMesh(
      core_axis_name="c", subcore_axis_name="s",
      num_cores=_NC, num_subcores=_NS)

  @functools.partial(
      pl.kernel,
      out_type=jax.ShapeDtypeStruct((B, D), jnp.float32),
      mesh=mesh,
      scratch_types=[
          pltpu.VMEM((c_per_w, _CHUNK), jnp.int32),
          pltpu.VMEM((_NBUF, _CHUNK, D), jnp.float32),
          pltpu.VMEM((b_per_w, D), jnp.float32),
      ] + [pltpu.SemaphoreType.DMA] * _NBUF,
      compiler_params=pltpu.CompilerParams(use_tc_tiling_on_sc=False),
  )
  def k(x2_hbm, table_hbm, out_hbm, idx_v, rows_v, out_v, *sems):
    wid = lax.axis_index("s") * _NC + lax.axis_index("c")
    cbase = wid * c_per_w
    bbase = wid * b_per_w

    pltpu.sync_copy(x2_hbm.at[pl.ds(cbase, c_per_w)], idx_v)

    for k0 in range(_NBUF):
      pltpu.async_copy(table_hbm.at[idx_v.at[k0]], rows_v.at[k0], sems[k0])

    def accum_chunk(buf, accs):
      def body(r4, accs):
        accs = list(accs)
        for u in range(4):
          r = r4 * 4 + u
          for d in range(nd):
            accs[d] = accs[d] + rows_v[buf, r, d * _L:(d + 1) * _L]
        return tuple(accs)
      return lax.fori_loop(0, _CHUNK // 4, body, accs)

    def pair_body(p, carry):
      for half in range(2):
        i = 2 * p + half
        accs = tuple(jnp.zeros((_L,), jnp.float32) for _ in range(nd))
        for k1 in (2 * half, 2 * half + 1):
          c = 4 * p + k1
          pltpu.make_async_copy(
              table_hbm.at[idx_v.at[c]], rows_v.at[k1], sems[k1]).wait()
          accs = accum_chunk(k1, accs)

          @pl.when(c + _NBUF < c_per_w)
          def _():
            pltpu.async_copy(
                table_hbm.at[idx_v.at[c + _NBUF]], rows_v.at[k1], sems[k1])

        for d in range(nd):
          out_v[i, d * _L:(d + 1) * _L] = accs[d]
      return carry

    lax.fori_loop(0, b_per_w // 2, pair_body, 0)
    pltpu.sync_copy(out_v, out_hbm.at[pl.ds(bbase, b_per_w)])

  return k(x2, table)


def _tc_relayout(tT):
  D, V = tT.shape
  CB = _CBREL
  NB = pl.cdiv(V, 2 * CB)

  def body(in_ref, out_ref):
    t = in_ref[...]
    stacked = jnp.concatenate([t[:, :CB], t[:, CB:]], axis=0)
    out_ref[...] = stacked.T

  return pl.pallas_call(
      body,
      grid=(NB,),
      in_specs=[pl.BlockSpec((D, 2 * CB), lambda i: (0, i))],
      out_specs=pl.BlockSpec((CB, 2 * D), lambda i: (i, 0)),
      out_shape=jax.ShapeDtypeStruct((NB * CB, 2 * D), jnp.float32),
  )(tT)


def _tc_head(x, pooled_sum, fc_w, fc_b2):
  B, S = x.shape
  D = pooled_sum.shape[1]
  C = fc_w.shape[0]
  BLK = 256
  assert B % BLK == 0

  def body(x_ref, ps_ref, w_ref, b_ref, out_ref):
    cnt = jnp.sum((x_ref[...] != 0).astype(jnp.float32), axis=1,
                  keepdims=True)
    pooled = jnp.maximum(ps_ref[...] / cnt, 0.0)
    out_ref[...] = lax.dot_general(
        w_ref[...], pooled, (((1,), (1,)), ((), ())),
        preferred_element_type=jnp.float32) + b_ref[...]

  return pl.pallas_call(
      body,
      grid=(B // BLK,),
      in_specs=[
          pl.BlockSpec((BLK, S), lambda i: (i, 0)),
          pl.BlockSpec((BLK, D), lambda i: (i, 0)),
          pl.BlockSpec((C, D), lambda i: (0, 0)),
          pl.BlockSpec((C, 1), lambda i: (0, 0)),
      ],
      out_specs=pl.BlockSpec((C, BLK), lambda i: (0, i)),
      out_shape=jax.ShapeDtypeStruct((C, B), jnp.float32),
  )(x, pooled_sum, fc_w, fc_b2)


def kernel(x, table, fc_w, fc_b):
  B, S = x.shape
  V, D = table.shape
  x = x.astype(jnp.int32)
  shift = _CBREL.bit_length() - 1
  q, r = x >> shift, x & (_CBREL - 1)
  px = (((q >> 1) << shift) + r) * 2 + (q & 1)
  x2 = px.reshape(2 * B, S // 2)
  tableT, x2 = jax.lax.optimization_barrier((table.T, x2))
  tableP = _tc_relayout(tableT)
  tableL = tableP.reshape(tableP.shape[0] * 2, D)
  pooled_sum = _sc_pooled_sum(x2, tableL)
  return _tc_head(x, pooled_sum, fc_w, fc_b.reshape(-1, 1)).T
